# Initial kernel scaffold; baseline (speedup 1.0000x reference)
import jax
import jax.numpy as jnp
from jax.experimental import pallas as pl


def kernel(x, edge_index, edge_weight, params):
    raise NotImplementedError("write your pallas kernel here")


# speedup vs baseline: 3.1008x; 3.1008x over previous
import functools

import jax
import jax.numpy as jnp
from jax import lax
from jax.experimental import pallas as pl
from jax.experimental.pallas import tpu as pltpu
from jax.experimental.pallas import tpu_sc as plsc

N = 10000
E = 320000
D = 128
NC = 2
NS = 16
LANES = 16
NW = NC * NS
EPW = E // NW
K = 80
NCHUNK = EPW // K
ZROWS = 200
NZCHUNK = N // ZROWS


def _spmm_sc(h, src, dst, wexp):
    mesh = plsc.VectorSubcoreMesh(core_axis_name="c", subcore_axis_name="s")

    @functools.partial(
        pl.kernel,
        out_type=jax.ShapeDtypeStruct((NC, N, D), jnp.float32),
        mesh=mesh,
        scratch_types=[
            pltpu.VMEM((K,), jnp.int32),
            pltpu.VMEM((K,), jnp.int32),
            pltpu.VMEM((K, LANES), jnp.float32),
            pltpu.VMEM((K, D), jnp.float32),
            pltpu.VMEM((ZROWS, D), jnp.float32),
            pltpu.VMEM_SHARED((N, D), jnp.float32),
            pltpu.SemaphoreType.DMA,
        ],
    )
    def k(h_hbm, src_hbm, dst_hbm, w_hbm, out_hbm,
          srcv, dstv, wv, rows, zbuf, acc, sem):
        cid = lax.axis_index("c")
        sid = lax.axis_index("s")

        zero = jnp.zeros((LANES,), jnp.float32)

        @pl.loop(0, ZROWS)
        def _(j):
            for r in range(D // LANES):
                zbuf[j, pl.ds(r * LANES, LANES)] = zero

        @pl.loop(0, (NZCHUNK + NS - 1) // NS)
        def _(t):
            c = sid + t * NS

            @pl.when(c < NZCHUNK)
            def _():
                pltpu.sync_copy(zbuf, acc.at[pl.ds(c * ZROWS, ZROWS)])

        plsc.subcore_barrier()

        base = (cid * NS + sid) * EPW

        @pl.loop(0, NCHUNK)
        def _(ck):
            off = base + ck * K
            pltpu.sync_copy(src_hbm.at[pl.ds(off, K)], srcv)
            pltpu.sync_copy(dst_hbm.at[pl.ds(off, K)], dstv)
            pltpu.sync_copy(w_hbm.at[pl.ds(off, K)], wv)
            pltpu.async_copy(h_hbm.at[srcv], rows, sem).wait()

            @pl.loop(0, K)
            def _(j):
                ws = wv[j]
                for r in range(D // LANES):
                    sl = pl.ds(r * LANES, LANES)
                    rows[j, sl] = rows[j, sl] * ws

            pltpu.sync_copy(rows, acc.at[dstv], add=True)

        plsc.subcore_barrier()

        @pl.loop(0, (NZCHUNK + NS - 1) // NS)
        def _(t):
            c = sid + t * NS

            @pl.when(c < NZCHUNK)
            def _():
                r0 = c * ZROWS
                pltpu.sync_copy(acc.at[pl.ds(r0, ZROWS)],
                                out_hbm.at[cid].at[pl.ds(r0, ZROWS)])

    return k(h, src, dst, wexp)


BN = 1000


def _dot(a, b):
    return lax.dot_general(a, b, (((1,), (0,)), ((), ())),
                           preferred_element_type=jnp.float32)


def _full_spec(shape):
    return pl.BlockSpec(shape, lambda i: tuple(0 for _ in shape))


def _init_tc(x, w_embed, b_embed, w_hinit, b_hinit):
    xr = x.shape[1]

    def body(x_ref, we, be, wh, bh, xres_ref, h_ref):
        xe = jax.nn.sigmoid(_dot(x_ref[...], we[...]) + be[...])
        xres_ref[...] = xe
        h_ref[...] = _dot(xe, wh[...]) + bh[...]

    return pl.pallas_call(
        body,
        grid=(N // BN,),
        in_specs=[
            pl.BlockSpec((BN, xr), lambda i: (i, 0)),
            _full_spec((xr, D)), _full_spec((1, D)),
            _full_spec((D, D)), _full_spec((1, D)),
        ],
        out_specs=[
            pl.BlockSpec((BN, D), lambda i: (i, 0)),
            pl.BlockSpec((BN, D), lambda i: (i, 0)),
        ],
        out_shape=[
            jax.ShapeDtypeStruct((N, D), jnp.float32),
            jax.ShapeDtypeStruct((N, D), jnp.float32),
        ],
    )(x, w_embed, b_embed.reshape(1, D), w_hinit, b_hinit.reshape(1, D))


def _gdu_tc(xres, zp, h, p, relu_z, final):
    dout = p['Wr'].shape[1]
    wf, we_, wt = p['Wf'], p['We'], p['Wt']
    ws = [wf[:D], wf[D:2 * D], wf[2 * D:],
          we_[:D], we_[D:2 * D], we_[2 * D:],
          wt[:D], wt[D:2 * D], wt[2 * D:], p['Wr']]
    bs = [p['bf'].reshape(1, D), p['be'].reshape(1, dout),
          p['bt'].reshape(1, dout), p['br'].reshape(1, dout)]

    def body(x_ref, z0_ref, z1_ref, h_ref,
             wfx, wfz, wfh, wex, wez, weh, wtx, wtz, wth, wr,
             bf, be, bt, br, o_ref):
        xv = x_ref[...]
        hv = h_ref[...]
        zv = z0_ref[...] + z1_ref[...]
        if relu_z:
            zv = jnp.maximum(zv, 0.0)
        f = jax.nn.sigmoid(_dot(xv, wfx[...]) + _dot(zv, wfz[...])
                           + _dot(hv, wfh[...]) + bf[...])
        e = jax.nn.sigmoid(_dot(xv, wex[...]) + _dot(zv, wez[...])
                           + _dot(hv, weh[...]) + be[...])
        t = jnp.tanh(_dot(xv, wtx[...]) + _dot(f * zv, wtz[...])
                     + _dot(hv, wth[...]) + bt[...])
        o = e * (_dot(hv, wr[...]) + br[...]) + (1.0 - e) * t
        if final:
            m = jnp.max(o, axis=1, keepdims=True)
            o = o - (m + jnp.log(jnp.sum(jnp.exp(o - m), axis=1, keepdims=True)))
        o_ref[...] = o

    w_specs = ([_full_spec((D, D))] * 3
               + [_full_spec((D, dout))] * 3
               + [_full_spec((D, dout))] * 3
               + [_full_spec((D, dout))])
    b_specs = [_full_spec((1, D)), _full_spec((1, dout)),
               _full_spec((1, dout)), _full_spec((1, dout))]
    w_specs[1] = _full_spec((D, D))
    w_specs[2] = _full_spec((D, D))

    return pl.pallas_call(
        body,
        grid=(N // BN,),
        in_specs=[pl.BlockSpec((BN, D), lambda i: (i, 0))] * 4 + w_specs + b_specs,
        out_specs=pl.BlockSpec((BN, dout), lambda i: (i, 0)),
        out_shape=jax.ShapeDtypeStruct((N, dout), jnp.float32),
    )(xres, zp[0], zp[1], h, *ws, *bs)


def kernel(x, edge_index, edge_weight, params):
    src = edge_index[1]
    dst = edge_index[0]
    wexp = jnp.broadcast_to(edge_weight[:, None], (E, LANES))
    wexp = jnp.asarray(wexp)

    xres, h = _init_tc(x, params['W_embed'], params['b_embed'],
                       params['W_hinit'], params['b_hinit'])
    zp = _spmm_sc(h, src, dst, wexp)
    h = _gdu_tc(xres, zp, h, params['gdu'][0], relu_z=False, final=False)
    for layer in range(1, 3):
        zp = _spmm_sc(h, src, dst, wexp)
        h = _gdu_tc(xres, zp, h, params['gdu'][layer], relu_z=True, final=False)
    zp = _spmm_sc(h, src, dst, wexp)
    return _gdu_tc(xres, zp, h, params['out'], relu_z=True, final=True)


# speedup vs baseline: 4.7713x; 1.5387x over previous
import functools

import jax
import jax.numpy as jnp
from jax import lax
from jax.experimental import pallas as pl
from jax.experimental.pallas import tpu as pltpu
from jax.experimental.pallas import tpu_sc as plsc

N = 10000
E = 320000
D = 128
NC = 2
NS = 16
LANES = 16
NW = NC * NS
EPW = E // NW
K = 80
NCHUNK = EPW // K
ZROWS = 200
NZCHUNK = N // ZROWS


NBUF = 4


def _spmm_sc(h, src3, dst3, wexp4, zeros):
    mesh = plsc.VectorSubcoreMesh(core_axis_name="c", subcore_axis_name="s")

    @functools.partial(
        pl.kernel,
        out_type=jax.ShapeDtypeStruct((NC, N, D), jnp.float32),
        mesh=mesh,
        scratch_types=(
            [pltpu.VMEM((K,), jnp.int32)] * NBUF
            + [pltpu.VMEM((K,), jnp.int32)] * NBUF
            + [pltpu.VMEM((K * LANES,), jnp.float32)] * NBUF
            + [pltpu.VMEM((K, D), jnp.float32)] * NBUF
            + [pltpu.VMEM_SHARED((N, D), jnp.float32)]
            + [pltpu.SemaphoreType.DMA] * (2 * NBUF)
        ),
    )
    def k(h_hbm, src_hbm, dst_hbm, w_hbm, z_hbm, out_hbm, *scr):
        srcv = scr[0:NBUF]
        dstv = scr[NBUF:2 * NBUF]
        wv = scr[2 * NBUF:3 * NBUF]
        rows = scr[3 * NBUF:4 * NBUF]
        acc = scr[4 * NBUF]
        gsem = scr[4 * NBUF + 1:5 * NBUF + 1]
        ssem = scr[5 * NBUF + 1:6 * NBUF + 1]

        cid = lax.axis_index("c")
        sid = lax.axis_index("s")
        wid = cid * NS + sid

        @pl.loop(0, (NZCHUNK + NS - 1) // NS)
        def _(t):
            c = sid + t * NS

            @pl.when(c < NZCHUNK)
            def _():
                sl = pl.ds(c * ZROWS, ZROWS)
                pltpu.sync_copy(z_hbm.at[sl], acc.at[sl])

        plsc.subcore_barrier()

        def gather_start(b, ck):
            pltpu.sync_copy(src_hbm.at[wid].at[ck], srcv[b])
            pltpu.sync_copy(dst_hbm.at[wid].at[ck], dstv[b])
            pltpu.sync_copy(w_hbm.at[wid].at[ck], wv[b])
            return pltpu.async_copy(h_hbm.at[srcv[b]], rows[b], gsem[b])

        def gather_wait(b, ck):
            pltpu.make_async_copy(h_hbm.at[srcv[b]], rows[b], gsem[b]).wait()

        def scatter_start(b, ck):
            return pltpu.async_copy(rows[b], acc.at[dstv[b]], ssem[b], add=True)

        def scatter_wait(b, ck):
            pltpu.make_async_copy(rows[b], acc.at[dstv[b]], ssem[b]).wait()

        def compute(b):
            @plsc.parallel_loop(0, K, unroll=4)
            def _(j):
                ws = wv[b][pl.ds(j * LANES, LANES)]
                for r in range(D // LANES):
                    sl = pl.ds(r * LANES, LANES)
                    rows[b][j, sl] = rows[b][j, sl] * ws

        for b in range(NBUF):
            gather_start(b, b)

        @pl.loop(0, (NCHUNK - 1) // NBUF)
        def _(i):
            for b in range(NBUF):
                ck = i * NBUF + b
                gather_wait(b, ck)
                compute(b)
                scatter_start(b, ck)
                nxt = ck + NBUF

                @pl.when(nxt < NCHUNK)
                def _():
                    scatter_wait(b, ck)
                    gather_start(b, nxt)

        last = NCHUNK - 1
        gather_wait(0, last)
        compute(0)
        pltpu.sync_copy(rows[0], acc.at[dstv[0]], add=True)
        for b in range(1, NBUF):
            scatter_wait(b, last - NBUF + b)

        plsc.subcore_barrier()

        @pl.loop(0, (NZCHUNK + NS - 1) // NS)
        def _(t):
            c = sid + t * NS

            @pl.when(c < NZCHUNK)
            def _():
                sl = pl.ds(c * ZROWS, ZROWS)
                pltpu.sync_copy(acc.at[sl], out_hbm.at[cid].at[sl])

    return k(h, src3, dst3, wexp4, zeros)


BN = 1000


def _dot(a, b):
    return lax.dot_general(a, b, (((1,), (0,)), ((), ())),
                           preferred_element_type=jnp.float32)


def _full_spec(shape):
    return pl.BlockSpec(shape, lambda i: tuple(0 for _ in shape))


def _init_tc(x, w_embed, b_embed, w_hinit, b_hinit):
    xr = x.shape[1]

    def body(x_ref, we, be, wh, bh, xres_ref, h_ref):
        xe = jax.nn.sigmoid(_dot(x_ref[...], we[...]) + be[...])
        xres_ref[...] = xe
        h_ref[...] = _dot(xe, wh[...]) + bh[...]

    return pl.pallas_call(
        body,
        grid=(N // BN,),
        in_specs=[
            pl.BlockSpec((BN, xr), lambda i: (i, 0)),
            _full_spec((xr, D)), _full_spec((1, D)),
            _full_spec((D, D)), _full_spec((1, D)),
        ],
        out_specs=[
            pl.BlockSpec((BN, D), lambda i: (i, 0)),
            pl.BlockSpec((BN, D), lambda i: (i, 0)),
        ],
        out_shape=[
            jax.ShapeDtypeStruct((N, D), jnp.float32),
            jax.ShapeDtypeStruct((N, D), jnp.float32),
        ],
    )(x, w_embed, b_embed.reshape(1, D), w_hinit, b_hinit.reshape(1, D))


def _gdu_tc(xres, zp, h, p, relu_z, final):
    dout = p['Wr'].shape[1]
    wf, we_, wt = p['Wf'], p['We'], p['Wt']
    ws = [wf[:D], wf[D:2 * D], wf[2 * D:],
          we_[:D], we_[D:2 * D], we_[2 * D:],
          wt[:D], wt[D:2 * D], wt[2 * D:], p['Wr']]
    bs = [p['bf'].reshape(1, D), p['be'].reshape(1, dout),
          p['bt'].reshape(1, dout), p['br'].reshape(1, dout)]

    def body(x_ref, z0_ref, z1_ref, h_ref,
             wfx, wfz, wfh, wex, wez, weh, wtx, wtz, wth, wr,
             bf, be, bt, br, o_ref):
        xv = x_ref[...]
        hv = h_ref[...]
        zv = z0_ref[...] + z1_ref[...]
        if relu_z:
            zv = jnp.maximum(zv, 0.0)
        f = jax.nn.sigmoid(_dot(xv, wfx[...]) + _dot(zv, wfz[...])
                           + _dot(hv, wfh[...]) + bf[...])
        e = jax.nn.sigmoid(_dot(xv, wex[...]) + _dot(zv, wez[...])
                           + _dot(hv, weh[...]) + be[...])
        t = jnp.tanh(_dot(xv, wtx[...]) + _dot(f * zv, wtz[...])
                     + _dot(hv, wth[...]) + bt[...])
        o = e * (_dot(hv, wr[...]) + br[...]) + (1.0 - e) * t
        if final:
            m = jnp.max(o, axis=1, keepdims=True)
            o = o - (m + jnp.log(jnp.sum(jnp.exp(o - m), axis=1, keepdims=True)))
        o_ref[...] = o

    w_specs = ([_full_spec((D, D))] * 3
               + [_full_spec((D, dout))] * 3
               + [_full_spec((D, dout))] * 3
               + [_full_spec((D, dout))])
    b_specs = [_full_spec((1, D)), _full_spec((1, dout)),
               _full_spec((1, dout)), _full_spec((1, dout))]
    w_specs[1] = _full_spec((D, D))
    w_specs[2] = _full_spec((D, D))

    return pl.pallas_call(
        body,
        grid=(N // BN,),
        in_specs=[pl.BlockSpec((BN, D), lambda i: (i, 0))] * 4 + w_specs + b_specs,
        out_specs=pl.BlockSpec((BN, dout), lambda i: (i, 0)),
        out_shape=jax.ShapeDtypeStruct((N, dout), jnp.float32),
    )(xres, zp[0], zp[1], h, *ws, *bs)


def kernel(x, edge_index, edge_weight, params):
    src3 = edge_index[1].reshape(NW, NCHUNK, K)
    dst3 = edge_index[0].reshape(NW, NCHUNK, K)
    wexp4 = jnp.broadcast_to(edge_weight.reshape(NW, NCHUNK, K)[..., None],
                             (NW, NCHUNK, K, LANES)).reshape(NW, NCHUNK, K * LANES)
    zeros = jnp.zeros((N, D), jnp.float32)

    xres, h = _init_tc(x, params['W_embed'], params['b_embed'],
                       params['W_hinit'], params['b_hinit'])
    zp = _spmm_sc(h, src3, dst3, wexp4, zeros)
    h = _gdu_tc(xres, zp, h, params['gdu'][0], relu_z=False, final=False)
    for layer in range(1, 3):
        zp = _spmm_sc(h, src3, dst3, wexp4, zeros)
        h = _gdu_tc(xres, zp, h, params['gdu'][layer], relu_z=True, final=False)
    zp = _spmm_sc(h, src3, dst3, wexp4, zeros)
    return _gdu_tc(xres, zp, h, params['out'], relu_z=True, final=True)
